# Initial kernel scaffold; baseline (speedup 1.0000x reference)
import jax
import jax.numpy as jnp
from jax.experimental import pallas as pl


def kernel(x, pos_embed):
    raise NotImplementedError("write your pallas kernel here")


# speedup vs baseline: 3.2960x; 3.2960x over previous
import jax
import jax.numpy as jnp
from jax.experimental import pallas as pl

S_BLK = 512


def _add_kernel(x_ref, pe_ref, out_ref):
    out_ref[...] = x_ref[...] + pe_ref[...][None, :, :]


def kernel(x, pos_embed):
    B, S, D = x.shape
    grid = (S // S_BLK,)
    return pl.pallas_call(
        _add_kernel,
        grid=grid,
        in_specs=[
            pl.BlockSpec((B, S_BLK, D), lambda i: (0, i, 0)),
            pl.BlockSpec((S_BLK, D), lambda i: (i, 0)),
        ],
        out_specs=pl.BlockSpec((B, S_BLK, D), lambda i: (0, i, 0)),
        out_shape=jax.ShapeDtypeStruct((B, S, D), x.dtype),
    )(x, pos_embed[:S])
